# baseline (speedup 1.0000x reference)
import functools
import math

import jax
import jax.numpy as jnp
from jax import lax
from jax.experimental import pallas as pl
from jax.experimental.pallas import tpu as pltpu
from jax.experimental.pallas import tpu_sc as plsc

_N = 720
_D = 64
_B = 524288
_TWO_PI = 2.0 * math.pi

_LANES = 128
_ROWS = _B // _LANES
_TC_BLOCK = 512

_NC, _NS = 2, 16
_NW = _NC * _NS
_C = 64
_NCHUNK = _B // (_NW * _C)
_HALF = _C // 2


def _tc_index_body(t_ref, iw_ref):
    x = t_ref[0]
    y = t_ref[1]
    z = t_ref[2]
    norm = jnp.sqrt(x * x + y * y + z * z) + 1e-8
    dx = x / norm
    dy = y / norm
    dz = z / norm
    dzc = jnp.clip(dz, -1.0 + 1e-6, 1.0 - 1e-6)
    theta = jnp.arctan2(jnp.sqrt((1.0 - dzc) * (1.0 + dzc)), dzc)
    phi = jnp.mod(jnp.arctan2(dy, dx), _TWO_PI)
    u = theta / _TWO_PI * _N
    v = phi / _TWO_PI * _N
    u0 = jnp.floor(u)
    v0 = jnp.floor(v)
    wu = u - u0
    wv = v - v0
    u0i = u0.astype(jnp.int32) % _N
    v0i = v0.astype(jnp.int32) % _N
    v1i = (v0i + 1) % _N
    i00 = u0i * _N + v0i
    i01 = u0i * _N + v1i
    iw_ref[:, 0, :] = i00 >> 1
    iw_ref[:, 1, :] = i01 >> 1
    iw_ref[:, 2, :] = (i00 & 1) * _D
    iw_ref[:, 3, :] = (i01 & 1) * _D
    bits = lambda a: lax.bitcast_convert_type(a, jnp.int32)
    iw_ref[:, 4, :] = bits((1.0 - wu) * (1.0 - wv))
    iw_ref[:, 5, :] = bits((1.0 - wu) * wv)
    iw_ref[:, 6, :] = bits(wu * (1.0 - wv))
    iw_ref[:, 7, :] = bits(wu * wv)


_tc_index = pl.pallas_call(
    _tc_index_body,
    grid=(_ROWS // _TC_BLOCK,),
    in_specs=[pl.BlockSpec((3, _TC_BLOCK, _LANES), lambda i: (0, i, 0))],
    out_specs=pl.BlockSpec((_TC_BLOCK, 8, _LANES), lambda i: (i, 0, 0)),
    out_shape=jax.ShapeDtypeStruct((_ROWS, 8, _LANES), jnp.int32),
)


def _sc_body(feat_hbm, iw_hbm, out_hbm, iw_v, idxb_v, rows_v, out_v,
             sem_idx, sem_g, sem_out):
    wid = lax.axis_index("s") * _NC + lax.axis_index("c")
    c0 = wid * _NCHUNK

    def meta_src(c):
        return iw_hbm.at[c // 2, :, pl.ds((c % 2) * _C, _C)]

    def gather_start(b):
        for cc in range(_C // 16):
            s = pl.ds(cc * 16, 16)
            idxb_v[b, 0, s] = iw_v[b, 0, s] + _N // 2
            idxb_v[b, 1, s] = iw_v[b, 1, s] + _N // 2
        for k in range(2):
            pltpu.make_async_copy(
                feat_hbm.at[iw_v.at[b, k]], rows_v.at[b, k], sem_g).start()
        for k in range(2):
            pltpu.make_async_copy(
                feat_hbm.at[idxb_v.at[b, k]], rows_v.at[b, 2 + k],
                sem_g).start()

    def gather_wait(b):
        for k in range(2):
            pltpu.make_async_copy(
                feat_hbm.at[iw_v.at[b, k]], rows_v.at[b, k], sem_g).wait()
        for k in range(2):
            pltpu.make_async_copy(
                feat_hbm.at[idxb_v.at[b, k]], rows_v.at[b, 2 + k],
                sem_g).wait()

    def meta_start(c, b):
        pltpu.make_async_copy(meta_src(c), iw_v.at[b], sem_idx).start()

    def meta_wait(c, b):
        pltpu.make_async_copy(meta_src(c), iw_v.at[b], sem_idx).wait()

    def out_start(c, b):
        pltpu.make_async_copy(
            out_v.at[b], out_hbm.at[pl.ds(c * _HALF, _HALF)], sem_out).start()

    def out_wait(c, b):
        pltpu.make_async_copy(
            out_v.at[b], out_hbm.at[pl.ds(c * _HALF, _HALF)], sem_out).wait()

    pltpu.sync_copy(meta_src(c0), iw_v.at[0])
    gather_start(0)
    meta_start(c0 + 1, 1)

    def blend(b):
        def body(gg, carry):
            base = gg * 16
            h0v = iw_v[b, 2, pl.ds(base, 16)]
            h1v = iw_v[b, 3, pl.ds(base, 16)]
            w00v = plsc.bitcast(iw_v[b, 4, pl.ds(base, 16)], jnp.float32)
            w01v = plsc.bitcast(iw_v[b, 5, pl.ds(base, 16)], jnp.float32)
            w10v = plsc.bitcast(iw_v[b, 6, pl.ds(base, 16)], jnp.float32)
            w11v = plsc.bitcast(iw_v[b, 7, pl.ds(base, 16)], jnp.float32)
            for l in range(16):
                j = base + l
                h0 = h0v[l]
                h1 = h1v[l]
                w00 = w00v[l]
                w01 = w01v[l]
                w10 = w10v[l]
                w11 = w11v[l]
                orow = gg * 8 + l // 2
                ocol = (l % 2) * _D
                for t in range(_D // 16):
                    acc = rows_v[b, 0, j, pl.ds(h0 + t * 16, 16)] * w00
                    acc = acc + rows_v[b, 1, j, pl.ds(h1 + t * 16, 16)] * w01
                    acc = acc + rows_v[b, 2, j, pl.ds(h0 + t * 16, 16)] * w10
                    acc = acc + rows_v[b, 3, j, pl.ds(h1 + t * 16, 16)] * w11
                    out_v[b, orow, pl.ds(ocol + t * 16, 16)] = acc
            return carry
        lax.fori_loop(0, _C // 16, body, 0)

    def g_body(g, carry):
        for b in range(2):
            i = 2 * g + b
            c = c0 + i
            gather_wait(b)

            @pl.when(i < _NCHUNK - 1)
            def _():
                meta_wait(c + 1, 1 - b)
                gather_start(1 - b)

            @pl.when(i >= 2)
            def _():
                out_wait(c - 2, b)

            blend(b)
            out_start(c, b)

            @pl.when(i < _NCHUNK - 2)
            def _():
                meta_start(c + 2, b)
        return carry

    lax.fori_loop(0, _NCHUNK // 2, g_body, 0)

    out_wait(c0 + _NCHUNK - 2, 0)
    out_wait(c0 + _NCHUNK - 1, 1)


@functools.cache
def _sc_gather_blend():
    return functools.partial(
        pl.kernel,
        out_type=jax.ShapeDtypeStruct((_B // 2, _LANES), jnp.float32),
        mesh=plsc.VectorSubcoreMesh(core_axis_name="c", subcore_axis_name="s",
                                    num_cores=_NC, num_subcores=_NS),
        scratch_types=[
            pltpu.VMEM((2, 8, _C), jnp.int32),
            pltpu.VMEM((2, 2, _C), jnp.int32),
            pltpu.VMEM((2, 4, _C, _LANES), jnp.float32),
            pltpu.VMEM((2, _HALF, _LANES), jnp.float32),
            pltpu.SemaphoreType.DMA,
            pltpu.SemaphoreType.DMA,
            pltpu.SemaphoreType.DMA,
        ],
        compiler_params=pltpu.CompilerParams(use_tc_tiling_on_sc=False,
                                             needs_layout_passes=False),
    )(_sc_body)


@jax.jit
def kernel(tgt, features):
    t3 = tgt.T.reshape(3, _ROWS, _LANES)
    iw = _tc_index(t3)
    feat2 = features.reshape(_N * _N // 2, 2 * _D)
    out2 = _sc_gather_blend()(feat2, iw)
    return out2.reshape(_B, _D)


# speedup vs baseline: 1.4393x; 1.4393x over previous
import functools
import math

import jax
import jax.numpy as jnp
from jax import lax
from jax.experimental import pallas as pl
from jax.experimental.pallas import tpu as pltpu
from jax.experimental.pallas import tpu_sc as plsc

_N = 720
_D = 64
_B = 524288
_TWO_PI = 2.0 * math.pi

_LANES = 128
_ROWS = _B // _LANES
_TC_BLOCK = 512

_NC, _NS = 2, 16
_NW = _NC * _NS
_C = 128
_NCHUNK = _B // (_NW * _C)

_TR_BLOCK = 512


def _tc_index_body(t_ref, iw_ref):
    x = t_ref[0]
    y = t_ref[1]
    z = t_ref[2]
    norm = jnp.sqrt(x * x + y * y + z * z) + 1e-8
    dx = x / norm
    dy = y / norm
    dz = z / norm
    dzc = jnp.clip(dz, -1.0 + 1e-6, 1.0 - 1e-6)
    theta = jnp.arctan2(jnp.sqrt((1.0 - dzc) * (1.0 + dzc)), dzc)
    phi = jnp.mod(jnp.arctan2(dy, dx), _TWO_PI)
    u = theta / _TWO_PI * _N
    v = phi / _TWO_PI * _N
    u0 = jnp.floor(u)
    v0 = jnp.floor(v)
    wu = u - u0
    wv = v - v0
    u0i = u0.astype(jnp.int32) % _N
    v0i = v0.astype(jnp.int32) % _N
    u1i = (u0i + 1) % _N
    v1i = (v0i + 1) % _N
    iw_ref[:, 0, :] = u0i * _N + v0i
    iw_ref[:, 1, :] = u0i * _N + v1i
    iw_ref[:, 2, :] = u1i * _N + v0i
    iw_ref[:, 3, :] = u1i * _N + v1i
    bits = lambda a: lax.bitcast_convert_type(a, jnp.int32)
    iw_ref[:, 4, :] = bits((1.0 - wu) * (1.0 - wv))
    iw_ref[:, 5, :] = bits((1.0 - wu) * wv)
    iw_ref[:, 6, :] = bits(wu * (1.0 - wv))
    iw_ref[:, 7, :] = bits(wu * wv)


_tc_index = pl.pallas_call(
    _tc_index_body,
    grid=(_ROWS // _TC_BLOCK,),
    in_specs=[pl.BlockSpec((3, _TC_BLOCK, _LANES), lambda i: (0, i, 0))],
    out_specs=pl.BlockSpec((_TC_BLOCK, 8, _LANES), lambda i: (i, 0, 0)),
    out_shape=jax.ShapeDtypeStruct((_ROWS, 8, _LANES), jnp.int32),
)


def _tc_transpose_body(x_ref, o_ref):
    x = x_ref[...]
    o_ref[:, 0:_TR_BLOCK] = x[:, 0:_D].T
    o_ref[:, _TR_BLOCK:2 * _TR_BLOCK] = x[:, _D:2 * _D].T


_tc_transpose = pl.pallas_call(
    _tc_transpose_body,
    grid=(_B // 2 // _TR_BLOCK,),
    in_specs=[pl.BlockSpec((_TR_BLOCK, _LANES), lambda i: (i, 0))],
    out_specs=pl.BlockSpec((_D, 2 * _TR_BLOCK), lambda i: (0, i)),
    out_shape=jax.ShapeDtypeStruct((_D, _B), jnp.float32),
)


def _sc_body(feat_hbm, iw_hbm, out_hbm, iw_v, rows_v, out_v,
             sem_idx, sem_g, sem_out):
    wid = lax.axis_index("s") * _NC + lax.axis_index("c")
    r0 = wid * _NCHUNK

    def gather_start(b):
        for k in range(4):
            pltpu.make_async_copy(
                feat_hbm.at[iw_v.at[b, k]], rows_v.at[b, k], sem_g).start()

    def gather_wait(b):
        for k in range(4):
            pltpu.make_async_copy(
                feat_hbm.at[iw_v.at[b, k]], rows_v.at[b, k], sem_g).wait()

    def meta_start(r, b):
        pltpu.make_async_copy(iw_hbm.at[r], iw_v.at[b], sem_idx).start()

    def meta_wait(r, b):
        pltpu.make_async_copy(iw_hbm.at[r], iw_v.at[b], sem_idx).wait()

    def out_dst(r):
        row0 = (r // 8) * 512 + (r % 4) * _C
        h = (r % 8) // 4
        return out_hbm.at[pl.ds(row0, _C), pl.ds(h * _D, _D)]

    def out_start(r, b):
        pltpu.make_async_copy(out_v.at[b], out_dst(r), sem_out).start()

    def out_wait(r, b):
        pltpu.make_async_copy(out_v.at[b], out_dst(r), sem_out).wait()

    pltpu.sync_copy(iw_hbm.at[r0], iw_v.at[0])
    gather_start(0)
    meta_start(r0 + 1, 1)

    def blend(b):
        def body(gg, carry):
            base = gg * 16
            w00v = plsc.bitcast(iw_v[b, 4, pl.ds(base, 16)], jnp.float32)
            w01v = plsc.bitcast(iw_v[b, 5, pl.ds(base, 16)], jnp.float32)
            w10v = plsc.bitcast(iw_v[b, 6, pl.ds(base, 16)], jnp.float32)
            w11v = plsc.bitcast(iw_v[b, 7, pl.ds(base, 16)], jnp.float32)
            for l in range(16):
                j = base + l
                w00 = w00v[l]
                w01 = w01v[l]
                w10 = w10v[l]
                w11 = w11v[l]
                for t in range(_D // 16):
                    s = pl.ds(t * 16, 16)
                    acc = rows_v[b, 0, j, s] * w00
                    acc = acc + rows_v[b, 1, j, s] * w01
                    acc = acc + rows_v[b, 2, j, s] * w10
                    acc = acc + rows_v[b, 3, j, s] * w11
                    out_v[b, j, s] = acc
            return carry
        lax.fori_loop(0, _C // 16, body, 0)

    def g_body(g, carry):
        for b in range(2):
            i = 2 * g + b
            r = r0 + i
            gather_wait(b)

            @pl.when(i < _NCHUNK - 1)
            def _():
                meta_wait(r + 1, 1 - b)
                gather_start(1 - b)

            @pl.when(i >= 2)
            def _():
                out_wait(r - 2, b)

            blend(b)
            out_start(r, b)

            @pl.when(i < _NCHUNK - 2)
            def _():
                meta_start(r + 2, b)
        return carry

    lax.fori_loop(0, _NCHUNK // 2, g_body, 0)

    out_wait(r0 + _NCHUNK - 2, 0)
    out_wait(r0 + _NCHUNK - 1, 1)


@functools.cache
def _sc_gather_blend():
    return functools.partial(
        pl.kernel,
        out_type=jax.ShapeDtypeStruct((_B // 2, _LANES), jnp.float32),
        mesh=plsc.VectorSubcoreMesh(core_axis_name="c", subcore_axis_name="s",
                                    num_cores=_NC, num_subcores=_NS),
        scratch_types=[
            pltpu.VMEM((2, 8, _C), jnp.int32),
            pltpu.VMEM((2, 4, _C, _D), jnp.float32),
            pltpu.VMEM((2, _C, _D), jnp.float32),
            pltpu.SemaphoreType.DMA,
            pltpu.SemaphoreType.DMA,
            pltpu.SemaphoreType.DMA,
        ],
        compiler_params=pltpu.CompilerParams(use_tc_tiling_on_sc=False,
                                             needs_layout_passes=False),
    )(_sc_body)


@jax.jit
def kernel(tgt, features):
    t3 = tgt.T.reshape(3, _ROWS, _LANES)
    iw = _tc_index(t3)
    feat2 = features.reshape(_N * _N, _D)
    out2 = _sc_gather_blend()(feat2, iw)
    out_t = _tc_transpose(out2)
    return out_t.T


# speedup vs baseline: 1.7660x; 1.2270x over previous
import functools
import math

import jax
import jax.numpy as jnp
from jax import lax
from jax.experimental import pallas as pl
from jax.experimental.pallas import tpu as pltpu
from jax.experimental.pallas import tpu_sc as plsc

_N = 720
_D = 64
_B = 524288
_TWO_PI = 2.0 * math.pi

_LANES = 128
_ROWS = _B // _LANES
_TC_BLOCK = 512

_NC, _NS = 2, 16
_NW = _NC * _NS
_C = 128
_NCHUNK = _B // (_NW * _C)

_TR_BLOCK = 2048


def _tc_index_body(t_ref, iw_ref):
    x = t_ref[0]
    y = t_ref[1]
    z = t_ref[2]
    norm = jnp.sqrt(x * x + y * y + z * z) + 1e-8
    dx = x / norm
    dy = y / norm
    dz = z / norm
    dzc = jnp.clip(dz, -1.0 + 1e-6, 1.0 - 1e-6)
    theta = jnp.arctan2(jnp.sqrt((1.0 - dzc) * (1.0 + dzc)), dzc)
    phi = jnp.mod(jnp.arctan2(dy, dx), _TWO_PI)
    u = theta / _TWO_PI * _N
    v = phi / _TWO_PI * _N
    u0 = jnp.floor(u)
    v0 = jnp.floor(v)
    wu = u - u0
    wv = v - v0
    u0i = u0.astype(jnp.int32) % _N
    v0i = v0.astype(jnp.int32) % _N
    u1i = (u0i + 1) % _N
    v1i = (v0i + 1) % _N
    iw_ref[:, 0, :] = u0i * _N + v0i
    iw_ref[:, 1, :] = u0i * _N + v1i
    iw_ref[:, 2, :] = u1i * _N + v0i
    iw_ref[:, 3, :] = u1i * _N + v1i
    bits = lambda a: lax.bitcast_convert_type(a, jnp.int32)
    iw_ref[:, 4, :] = bits((1.0 - wu) * (1.0 - wv))
    iw_ref[:, 5, :] = bits((1.0 - wu) * wv)
    iw_ref[:, 6, :] = bits(wu * (1.0 - wv))
    iw_ref[:, 7, :] = bits(wu * wv)


_tc_index = pl.pallas_call(
    _tc_index_body,
    grid=(_ROWS // _TC_BLOCK,),
    in_specs=[pl.BlockSpec((3, _TC_BLOCK, _LANES), lambda i: (0, i, 0))],
    out_specs=pl.BlockSpec((_TC_BLOCK, 8, _LANES), lambda i: (i, 0, 0)),
    out_shape=jax.ShapeDtypeStruct((_ROWS, 8, _LANES), jnp.int32),
)


def _tc_transpose_body(x_ref, o_ref):
    x = x_ref[...]
    o_ref[:, 0:_TR_BLOCK] = x[:, 0:_D].T
    o_ref[:, _TR_BLOCK:2 * _TR_BLOCK] = x[:, _D:2 * _D].T


_tc_transpose = pl.pallas_call(
    _tc_transpose_body,
    grid=(_B // 2 // _TR_BLOCK,),
    in_specs=[pl.BlockSpec((_TR_BLOCK, _LANES), lambda i: (i, 0))],
    out_specs=pl.BlockSpec((_D, 2 * _TR_BLOCK), lambda i: (0, i)),
    out_shape=jax.ShapeDtypeStruct((_D, _B), jnp.float32),
)


def _sc_body(feat_hbm, iw_hbm, out_hbm, iw_v, rows_v, out_v,
             sem_idx, sem_g, sem_out):
    wid = lax.axis_index("s") * _NC + lax.axis_index("c")
    r0 = wid * _NCHUNK

    def gather_start(b):
        for k in range(4):
            pltpu.make_async_copy(
                feat_hbm.at[iw_v.at[b, k]], rows_v.at[b, k], sem_g).start()

    def gather_wait(b):
        for k in range(4):
            pltpu.make_async_copy(
                feat_hbm.at[iw_v.at[b, k]], rows_v.at[b, k], sem_g).wait()

    def meta_start(r, b):
        pltpu.make_async_copy(iw_hbm.at[r], iw_v.at[b], sem_idx).start()

    def meta_wait(r, b):
        pltpu.make_async_copy(iw_hbm.at[r], iw_v.at[b], sem_idx).wait()

    def out_dst(r):
        cpb = _TR_BLOCK // _C
        row0 = (r // (2 * cpb)) * _TR_BLOCK + (r % cpb) * _C
        h = (r % (2 * cpb)) // cpb
        return out_hbm.at[pl.ds(row0, _C), pl.ds(h * _D, _D)]

    def out_start(r, b):
        pltpu.make_async_copy(out_v.at[b], out_dst(r), sem_out).start()

    def out_wait(r, b):
        pltpu.make_async_copy(out_v.at[b], out_dst(r), sem_out).wait()

    pltpu.sync_copy(iw_hbm.at[r0], iw_v.at[0])
    gather_start(0)
    meta_start(r0 + 1, 1)

    def blend(b):
        def body(gg, carry):
            base = gg * 16
            w00v = plsc.bitcast(iw_v[b, 4, pl.ds(base, 16)], jnp.float32)
            w01v = plsc.bitcast(iw_v[b, 5, pl.ds(base, 16)], jnp.float32)
            w10v = plsc.bitcast(iw_v[b, 6, pl.ds(base, 16)], jnp.float32)
            w11v = plsc.bitcast(iw_v[b, 7, pl.ds(base, 16)], jnp.float32)
            for l in range(16):
                j = base + l
                w00 = w00v[l]
                w01 = w01v[l]
                w10 = w10v[l]
                w11 = w11v[l]
                for t in range(_D // 16):
                    s = pl.ds(t * 16, 16)
                    acc = rows_v[b, 0, j, s] * w00
                    acc = acc + rows_v[b, 1, j, s] * w01
                    acc = acc + rows_v[b, 2, j, s] * w10
                    acc = acc + rows_v[b, 3, j, s] * w11
                    out_v[b, j, s] = acc
            return carry
        lax.fori_loop(0, _C // 16, body, 0)

    def g_body(g, carry):
        for b in range(2):
            i = 2 * g + b
            r = r0 + i
            gather_wait(b)

            @pl.when(i < _NCHUNK - 1)
            def _():
                meta_wait(r + 1, 1 - b)
                gather_start(1 - b)

            @pl.when(i >= 2)
            def _():
                out_wait(r - 2, b)

            blend(b)
            out_start(r, b)

            @pl.when(i < _NCHUNK - 2)
            def _():
                meta_start(r + 2, b)
        return carry

    lax.fori_loop(0, _NCHUNK // 2, g_body, 0)

    out_wait(r0 + _NCHUNK - 2, 0)
    out_wait(r0 + _NCHUNK - 1, 1)


@functools.cache
def _sc_gather_blend():
    return functools.partial(
        pl.kernel,
        out_type=jax.ShapeDtypeStruct((_B // 2, _LANES), jnp.float32),
        mesh=plsc.VectorSubcoreMesh(core_axis_name="c", subcore_axis_name="s",
                                    num_cores=_NC, num_subcores=_NS),
        scratch_types=[
            pltpu.VMEM((2, 8, _C), jnp.int32),
            pltpu.VMEM((2, 4, _C, _D), jnp.float32),
            pltpu.VMEM((2, _C, _D), jnp.float32),
            pltpu.SemaphoreType.DMA,
            pltpu.SemaphoreType.DMA,
            pltpu.SemaphoreType.DMA,
        ],
        compiler_params=pltpu.CompilerParams(use_tc_tiling_on_sc=False,
                                             needs_layout_passes=False),
    )(_sc_body)


@jax.jit
def kernel(tgt, features):
    t3 = tgt.T.reshape(3, _ROWS, _LANES)
    iw = _tc_index(t3)
    feat2 = features.reshape(_N * _N, _D)
    out2 = _sc_gather_blend()(feat2, iw)
    out_t = _tc_transpose(out2)
    return out_t.T


# speedup vs baseline: 1.7838x; 1.0101x over previous
import functools
import math

import jax
import jax.numpy as jnp
from jax import lax
from jax.experimental import pallas as pl
from jax.experimental.pallas import tpu as pltpu
from jax.experimental.pallas import tpu_sc as plsc

_N = 720
_D = 64
_B = 524288
_TWO_PI = 2.0 * math.pi

_LANES = 128
_ROWS = _B // _LANES
_TC_BLOCK = 512

_NC, _NS = 2, 16
_NW = _NC * _NS
_C = 128
_NCHUNK = _B // (_NW * _C)

_TR_BLOCK = 2048


def _tc_index_body(t_ref, iw_ref):
    x = t_ref[0]
    y = t_ref[1]
    z = t_ref[2]
    norm = jnp.sqrt(x * x + y * y + z * z) + 1e-8
    dx = x / norm
    dy = y / norm
    dz = z / norm
    dzc = jnp.clip(dz, -1.0 + 1e-6, 1.0 - 1e-6)
    theta = jnp.arctan2(jnp.sqrt((1.0 - dzc) * (1.0 + dzc)), dzc)
    phi = jnp.mod(jnp.arctan2(dy, dx), _TWO_PI)
    u = theta / _TWO_PI * _N
    v = phi / _TWO_PI * _N
    u0 = jnp.floor(u)
    v0 = jnp.floor(v)
    wu = u - u0
    wv = v - v0
    u0i = u0.astype(jnp.int32) % _N
    v0i = v0.astype(jnp.int32) % _N
    u1i = (u0i + 1) % _N
    v1i = (v0i + 1) % _N

    def perm(i):
        blk = i // 2880
        pos = i % 2880
        return blk * 2880 + 2 * (pos % 1440) + pos // 1440

    iw_ref[:, 0, :] = perm(u0i * _N + v0i)
    iw_ref[:, 1, :] = perm(u0i * _N + v1i)
    iw_ref[:, 2, :] = perm(u1i * _N + v0i)
    iw_ref[:, 3, :] = perm(u1i * _N + v1i)
    bits = lambda a: lax.bitcast_convert_type(a, jnp.int32)
    iw_ref[:, 4, :] = bits((1.0 - wu) * (1.0 - wv))
    iw_ref[:, 5, :] = bits((1.0 - wu) * wv)
    iw_ref[:, 6, :] = bits(wu * (1.0 - wv))
    iw_ref[:, 7, :] = bits(wu * wv)


_tc_index = pl.pallas_call(
    _tc_index_body,
    grid=(_ROWS // _TC_BLOCK,),
    in_specs=[pl.BlockSpec((3, _TC_BLOCK, _LANES), lambda i: (0, i, 0))],
    out_specs=pl.BlockSpec((_TC_BLOCK, 8, _LANES), lambda i: (i, 0, 0)),
    out_shape=jax.ShapeDtypeStruct((_ROWS, 8, _LANES), jnp.int32),
)


def _tc_transpose_body(x_ref, o_ref):
    x = x_ref[...]
    o_ref[:, 0:_TR_BLOCK] = x[:, 0:_D].T
    o_ref[:, _TR_BLOCK:2 * _TR_BLOCK] = x[:, _D:2 * _D].T


_tc_transpose = pl.pallas_call(
    _tc_transpose_body,
    grid=(_B // 2 // _TR_BLOCK,),
    in_specs=[pl.BlockSpec((_TR_BLOCK, _LANES), lambda i: (i, 0))],
    out_specs=pl.BlockSpec((_D, 2 * _TR_BLOCK), lambda i: (0, i)),
    out_shape=jax.ShapeDtypeStruct((_D, _B), jnp.float32),
)


_DP_BLOCK = 2880


def _tc_depad_body(x_ref, o_ref):
    x = x_ref[...]
    h = _DP_BLOCK // 2
    o_ref[...] = jnp.concatenate([x[0:h, :], x[h:_DP_BLOCK, :]], axis=1)


_tc_depad = pl.pallas_call(
    _tc_depad_body,
    grid=(_N * _N // _DP_BLOCK,),
    in_specs=[pl.BlockSpec((_DP_BLOCK, _D), lambda i: (i, 0))],
    out_specs=pl.BlockSpec((_DP_BLOCK // 2, 2 * _D), lambda i: (i, 0)),
    out_shape=jax.ShapeDtypeStruct((_N * _N // 2, 2 * _D), jnp.float32),
)


def _sc_body(feat_hbm, iw_hbm, out_hbm, iw_v, rows_v, out_v,
             sem_idx, sem_g, sem_out):
    wid = lax.axis_index("s") * _NC + lax.axis_index("c")
    r0 = wid * _NCHUNK

    def gather_start(b):
        for k in range(4):
            pltpu.make_async_copy(
                feat_hbm.at[iw_v.at[b, k]], rows_v.at[b, k], sem_g).start()

    def gather_wait(b):
        for k in range(4):
            pltpu.make_async_copy(
                feat_hbm.at[iw_v.at[b, k]], rows_v.at[b, k], sem_g).wait()

    def meta_start(r, b):
        pltpu.make_async_copy(iw_hbm.at[r], iw_v.at[b], sem_idx).start()

    def meta_wait(r, b):
        pltpu.make_async_copy(iw_hbm.at[r], iw_v.at[b], sem_idx).wait()

    def out_dst(r):
        cpb = _TR_BLOCK // _C
        row0 = (r // (2 * cpb)) * _TR_BLOCK + (r % cpb) * _C
        h = (r % (2 * cpb)) // cpb
        return out_hbm.at[pl.ds(row0, _C), pl.ds(h * _D, _D)]

    def out_start(r, b):
        pltpu.make_async_copy(out_v.at[b], out_dst(r), sem_out).start()

    def out_wait(r, b):
        pltpu.make_async_copy(out_v.at[b], out_dst(r), sem_out).wait()

    pltpu.sync_copy(iw_hbm.at[r0], iw_v.at[0])
    gather_start(0)
    meta_start(r0 + 1, 1)

    def blend(b):
        def body(gg, carry):
            base = gg * 16
            w00v = plsc.bitcast(iw_v[b, 4, pl.ds(base, 16)], jnp.float32)
            w01v = plsc.bitcast(iw_v[b, 5, pl.ds(base, 16)], jnp.float32)
            w10v = plsc.bitcast(iw_v[b, 6, pl.ds(base, 16)], jnp.float32)
            w11v = plsc.bitcast(iw_v[b, 7, pl.ds(base, 16)], jnp.float32)
            for l in range(16):
                j = base + l
                w00 = w00v[l]
                w01 = w01v[l]
                w10 = w10v[l]
                w11 = w11v[l]
                for t in range(_D // 16):
                    s = pl.ds(t * 16, 16)
                    acc = rows_v[b, 0, j, s] * w00
                    acc = acc + rows_v[b, 1, j, s] * w01
                    acc = acc + rows_v[b, 2, j, s] * w10
                    acc = acc + rows_v[b, 3, j, s] * w11
                    out_v[b, j, s] = acc
            return carry
        lax.fori_loop(0, _C // 16, body, 0)

    def g_body(g, carry):
        for b in range(2):
            i = 2 * g + b
            r = r0 + i
            gather_wait(b)

            @pl.when(i < _NCHUNK - 1)
            def _():
                meta_wait(r + 1, 1 - b)
                gather_start(1 - b)

            @pl.when(i >= 2)
            def _():
                out_wait(r - 2, b)

            blend(b)
            out_start(r, b)

            @pl.when(i < _NCHUNK - 2)
            def _():
                meta_start(r + 2, b)
        return carry

    lax.fori_loop(0, _NCHUNK // 2, g_body, 0)

    out_wait(r0 + _NCHUNK - 2, 0)
    out_wait(r0 + _NCHUNK - 1, 1)


@functools.cache
def _sc_gather_blend():
    return functools.partial(
        pl.kernel,
        out_type=jax.ShapeDtypeStruct((_B // 2, _LANES), jnp.float32),
        mesh=plsc.VectorSubcoreMesh(core_axis_name="c", subcore_axis_name="s",
                                    num_cores=_NC, num_subcores=_NS),
        scratch_types=[
            pltpu.VMEM((2, 8, _C), jnp.int32),
            pltpu.VMEM((2, 4, _C, _D), jnp.float32),
            pltpu.VMEM((2, _C, _D), jnp.float32),
            pltpu.SemaphoreType.DMA,
            pltpu.SemaphoreType.DMA,
            pltpu.SemaphoreType.DMA,
        ],
        compiler_params=pltpu.CompilerParams(use_tc_tiling_on_sc=False,
                                             needs_layout_passes=False),
    )(_sc_body)


@jax.jit
def kernel(tgt, features):
    t3 = tgt.T.reshape(3, _ROWS, _LANES)
    iw = _tc_index(t3)
    feat_c = _tc_depad(features.reshape(_N * _N, _D))
    out2 = _sc_gather_blend()(feat_c.reshape(_N * _N, _D), iw)
    out_t = _tc_transpose(out2)
    return out_t.T
